# baseline (speedup 1.0000x reference)
import functools

import jax
import jax.numpy as jnp
from jax import lax
from jax.experimental import pallas as pl
from jax.experimental.pallas import tpu as pltpu
from jax.experimental.pallas import tpu_sc as plsc

B = 2
T = 8
NUM_NODES = 10000
E = 320000
H = 64
HEADS = 4
GAT_DIM = 128
N2 = B * NUM_NODES


RL = 1000


def _lstm_body(x_ref, wih0f, whh0f, b0f, b0f2, wih0b, whh0b, b0b, b0b2,
               wih1f, whh1f, b1f, b1f2, wih1b, whh1b, b1b, b1b2, out_ref):
    x = x_ref[...]

    def cell(pre, h, c, whhT, bih, bhh):
        gates = ((pre + jnp.dot(h, whhT[...], preferred_element_type=jnp.float32))
                 + bih[...]) + bhh[...]
        i = gates[:, 0:H]
        f = gates[:, H:2 * H]
        g = gates[:, 2 * H:3 * H]
        o = gates[:, 3 * H:4 * H]
        c = jax.nn.sigmoid(f) * c + jax.nn.sigmoid(i) * jnp.tanh(g)
        h = jax.nn.sigmoid(o) * jnp.tanh(c)
        return h, c

    z = jnp.zeros((x.shape[0], H), jnp.float32)

    hf0 = []
    h, c = z, z
    for t in range(T):
        h, c = cell(x[:, t:t + 1] * wih0f[...], h, c, whh0f, b0f, b0f2)
        hf0.append(h)
    hb0 = [None] * T
    h, c = z, z
    for t in range(T - 1, -1, -1):
        h, c = cell(x[:, t:t + 1] * wih0b[...], h, c, whh0b, b0b, b0b2)
        hb0[t] = h

    zs = [jnp.concatenate([hf0[t], hb0[t]], axis=1) for t in range(T)]

    sumf = z
    h, c = z, z
    for t in range(T):
        h, c = cell(jnp.dot(zs[t], wih1f[...], preferred_element_type=jnp.float32),
                    h, c, whh1f, b1f, b1f2)
        sumf = sumf + h
    hb1 = [None] * T
    h, c = z, z
    for t in range(T - 1, -1, -1):
        h, c = cell(jnp.dot(zs[t], wih1b[...], preferred_element_type=jnp.float32),
                    h, c, whh1b, b1b, b1b2)
        hb1[t] = h
    sumb = z
    for t in range(T):
        sumb = sumb + hb1[t]

    out_ref[...] = jnp.concatenate([sumf, sumb], axis=1) * (1.0 / T)


def _lstm_call(xs2d, wp):
    full = lambda shp: pl.BlockSpec(shp, lambda i: (0, 0))
    specs = [pl.BlockSpec((RL, T), lambda i: (i, 0))]
    for l in (0, 1):
        for d in ('f', 'b'):
            ih_shape = (1, 4 * H) if l == 0 else (2 * H, 4 * H)
            specs += [full(ih_shape), full((H, 4 * H)), full((1, 4 * H)),
                      full((1, 4 * H))]
    args = [xs2d]
    for l in (0, 1):
        for d in ('f', 'b'):
            pre = 'lstm%d%s_' % (l, d)
            args += [wp[pre + 'Wih'].T, wp[pre + 'Whh'].T,
                     wp[pre + 'bih'][None, :], wp[pre + 'bhh'][None, :]]
    return pl.pallas_call(
        _lstm_body,
        grid=(N2 // RL,),
        in_specs=specs,
        out_specs=pl.BlockSpec((RL, GAT_DIM), lambda i: (i, 0)),
        out_shape=jax.ShapeDtypeStruct((N2, GAT_DIM), jnp.float32),
    )(*args)


RP = 2000


def _proj_body(h_ref, wT_ref, asf_ref, adf_ref, hp_ref, asp_ref, adp_ref):
    hp = jnp.dot(h_ref[...], wT_ref[...], preferred_element_type=jnp.float32,
                 precision=lax.Precision.HIGHEST)
    hp_ref[...] = hp
    ts = hp * asf_ref[...]
    td = hp * adf_ref[...]
    z = jnp.zeros((hp.shape[0], 12), jnp.float32)
    cols = [jnp.sum(ts[:, 32 * k:32 * (k + 1)], axis=1, keepdims=True)
            for k in range(HEADS)] + [z]
    asp_ref[...] = jnp.concatenate(cols, axis=1)
    cols = [jnp.sum(td[:, 32 * k:32 * (k + 1)], axis=1, keepdims=True)
            for k in range(HEADS)] + [z]
    adp_ref[...] = jnp.concatenate(cols, axis=1)


def _proj_call(h, W, att_src, att_dst):
    full = lambda shp: pl.BlockSpec(shp, lambda i: (0, 0))
    return pl.pallas_call(
        _proj_body,
        grid=(N2 // RP,),
        in_specs=[pl.BlockSpec((RP, GAT_DIM), lambda i: (i, 0)),
                  full((GAT_DIM, GAT_DIM)), full((1, GAT_DIM)), full((1, GAT_DIM))],
        out_specs=[pl.BlockSpec((RP, GAT_DIM), lambda i: (i, 0)),
                   pl.BlockSpec((RP, 16), lambda i: (i, 0)),
                   pl.BlockSpec((RP, 16), lambda i: (i, 0))],
        out_shape=[jax.ShapeDtypeStruct((N2, GAT_DIM), jnp.float32),
                   jax.ShapeDtypeStruct((N2, 16), jnp.float32),
                   jax.ShapeDtypeStruct((N2, 16), jnp.float32)],
    )(h, W.T, att_src.reshape(1, GAT_DIM), att_dst.reshape(1, GAT_DIM))


CH = 80
SUP = 8
NROW = E // CH
NSUPT = NROW // SUP
ZB = 80
NZB = NUM_NODES // ZB
ACC_W = 144


def _edge_sc_body(hp0, hp1, as0, as1, ad0, ad1, src2d, dst2d, out_hbm,
                  srcb, dstb, gbuf, asb, adb, rows,
                  acc, gsem, asem, adsem):
    c = lax.axis_index("c")
    s = lax.axis_index("s")
    z16 = jnp.zeros((16,), jnp.float32)

    @pl.loop(0, ZB)
    def _zero(r):
        for f in range(ACC_W // 16):
            rows[r, pl.ds(16 * f, 16)] = z16

    @pl.loop(0, (NZB + 15) // 16)
    def _zcp(q):
        b = s + 16 * q

        @pl.when(b < NZB)
        def _():
            pltpu.sync_copy(rows, acc.at[pl.ds(ZB * b, ZB)])

    plsc.subcore_barrier()

    def compute():
        @pl.loop(0, CH, unroll=4)
        def _edge(e):
            w = asb[e, pl.ds(0, 16)] + adb[e, pl.ds(0, 16)]
            w = jnp.where(w >= 0.0, w, w * 0.2)
            w = jnp.exp(w)
            rows[e, pl.ds(GAT_DIM, 16)] = w
            ws = (w[0], w[1], w[2], w[3])
            for f in range(8):
                rows[e, pl.ds(16 * f, 16)] = gbuf[e, pl.ds(16 * f, 16)] * ws[f // 2]

    @pl.loop(0, (NSUPT + 15) // 16)
    def _sup(j):
        u = s + 16 * j

        @pl.when(u < NSUPT)
        def _():
            row0 = SUP * u
            pltpu.sync_copy(src2d.at[pl.ds(row0, SUP)], srcb)
            pltpu.sync_copy(dst2d.at[pl.ds(row0, SUP)], dstb)

            @pl.loop(0, SUP)
            def _chunk(i):
                @pl.when(c == 0)
                def _():
                    d1 = pltpu.async_copy(hp0.at[srcb.at[i]], gbuf, gsem)
                    d2 = pltpu.async_copy(as0.at[srcb.at[i]], asb, asem)
                    d3 = pltpu.async_copy(ad0.at[dstb.at[i]], adb, adsem)
                    d1.wait()
                    d2.wait()
                    d3.wait()

                @pl.when(c == 1)
                def _():
                    d1 = pltpu.async_copy(hp1.at[srcb.at[i]], gbuf, gsem)
                    d2 = pltpu.async_copy(as1.at[srcb.at[i]], asb, asem)
                    d3 = pltpu.async_copy(ad1.at[dstb.at[i]], adb, adsem)
                    d1.wait()
                    d2.wait()
                    d3.wait()

                compute()
                pltpu.sync_copy(rows, acc.at[dstb.at[i]], add=True)

    plsc.subcore_barrier()

    @pl.loop(0, (NZB + 15) // 16)
    def _wout(q):
        b = s + 16 * q

        @pl.when(b < NZB)
        def _():
            pltpu.sync_copy(acc.at[pl.ds(ZB * b, ZB)],
                            out_hbm.at[pl.ds(c * NUM_NODES + ZB * b, ZB)])


def _edge_call(hp, asp, adp, src2d, dst2d):
    mesh = plsc.VectorSubcoreMesh(core_axis_name="c", subcore_axis_name="s",
                                  num_cores=2, num_subcores=16)
    fn = pl.kernel(
        _edge_sc_body,
        out_type=jax.ShapeDtypeStruct((N2, ACC_W), jnp.float32),
        mesh=mesh,
        compiler_params=pltpu.CompilerParams(use_tc_tiling_on_sc=False),
        scratch_types=[
            pltpu.VMEM((SUP, CH), jnp.int32),
            pltpu.VMEM((SUP, CH), jnp.int32),
            pltpu.VMEM((CH, GAT_DIM), jnp.float32),
            pltpu.VMEM((CH, 16), jnp.float32),
            pltpu.VMEM((CH, 16), jnp.float32),
            pltpu.VMEM((CH, ACC_W), jnp.float32),
            pltpu.VMEM_SHARED((NUM_NODES, ACC_W), jnp.float32),
            pltpu.SemaphoreType.DMA,
            pltpu.SemaphoreType.DMA,
            pltpu.SemaphoreType.DMA,
        ],
    )
    return fn(hp[:NUM_NODES], hp[NUM_NODES:], asp[:NUM_NODES], asp[NUM_NODES:],
              adp[:NUM_NODES], adp[NUM_NODES:], src2d, dst2d)


RA = 2000
BPB = NUM_NODES // RA


def _postA_body(o_ref, bias_ref, y_ref, s1_ref, s2_ref):
    o = o_ref[...]
    cols = []
    for k in range(HEADS):
        cols.append(o[:, 32 * k:32 * (k + 1)]
                    / (o[:, GAT_DIM + k:GAT_DIM + k + 1] + 1e-16))
    y = jnp.concatenate(cols, axis=1) + bias_ref[...]
    y_ref[...] = y

    @pl.when(pl.program_id(0) % BPB == 0)
    def _():
        s1_ref[...] = jnp.zeros_like(s1_ref)
        s2_ref[...] = jnp.zeros_like(s2_ref)

    s1_ref[...] += jnp.sum(y, axis=0, keepdims=True)[None]
    s2_ref[...] += jnp.sum(y * y, axis=0, keepdims=True)[None]


def _postA_call(o, bias):
    full = lambda shp: pl.BlockSpec(shp, lambda i: (0, 0))
    return pl.pallas_call(
        _postA_body,
        grid=(N2 // RA,),
        in_specs=[pl.BlockSpec((RA, ACC_W), lambda i: (i, 0)), full((1, GAT_DIM))],
        out_specs=[pl.BlockSpec((RA, GAT_DIM), lambda i: (i, 0)),
                   pl.BlockSpec((1, 1, GAT_DIM), lambda i: (i // BPB, 0, 0)),
                   pl.BlockSpec((1, 1, GAT_DIM), lambda i: (i // BPB, 0, 0))],
        out_shape=[jax.ShapeDtypeStruct((N2, GAT_DIM), jnp.float32),
                   jax.ShapeDtypeStruct((B, 1, GAT_DIM), jnp.float32),
                   jax.ShapeDtypeStruct((B, 1, GAT_DIM), jnp.float32)],
    )(o, bias.reshape(1, GAT_DIM))


def _postB_compute(y_ref, res_ref, s1_ref, s2_ref, w_ref, b_ref, ms_ref):
    inv_n = 1.0 / NUM_NODES
    mean = s1_ref[0] * inv_n
    mm = mean * ms_ref[...]
    var = s2_ref[0] * inv_n - 2.0 * mm * mean + mm * mm
    inv = 1.0 / jnp.sqrt(var + 1e-5)
    xc = y_ref[...] - mm
    return jax.nn.relu(w_ref[...] * xc * inv + b_ref[...] + res_ref[...])


def _postB_body(y_ref, res_ref, s1_ref, s2_ref, w_ref, b_ref, ms_ref, out_ref):
    out_ref[...] = _postB_compute(y_ref, res_ref, s1_ref, s2_ref, w_ref, b_ref, ms_ref)


def _postB_pool_body(y_ref, res_ref, s1_ref, s2_ref, w_ref, b_ref, ms_ref,
                     out_ref, p_ref):
    h = _postB_compute(y_ref, res_ref, s1_ref, s2_ref, w_ref, b_ref, ms_ref)
    out_ref[...] = h

    @pl.when(pl.program_id(0) % BPB == 0)
    def _():
        p_ref[...] = jnp.zeros_like(p_ref)

    p_ref[...] += jnp.sum(h, axis=0, keepdims=True)[None]


def _postB_call(y, res, s1, s2, w, b, ms, with_pool):
    full = lambda shp: pl.BlockSpec(shp, lambda i: (0, 0))
    in_specs = [pl.BlockSpec((RA, GAT_DIM), lambda i: (i, 0)),
                pl.BlockSpec((RA, GAT_DIM), lambda i: (i, 0)),
                pl.BlockSpec((1, 1, GAT_DIM), lambda i: (i // BPB, 0, 0)),
                pl.BlockSpec((1, 1, GAT_DIM), lambda i: (i // BPB, 0, 0)),
                full((1, GAT_DIM)), full((1, GAT_DIM)), full((1, GAT_DIM))]
    out_specs = [pl.BlockSpec((RA, GAT_DIM), lambda i: (i, 0))]
    out_shape = [jax.ShapeDtypeStruct((N2, GAT_DIM), jnp.float32)]
    body = _postB_body
    if with_pool:
        body = _postB_pool_body
        out_specs.append(pl.BlockSpec((1, 1, GAT_DIM), lambda i: (i // BPB, 0, 0)))
        out_shape.append(jax.ShapeDtypeStruct((B, 1, GAT_DIM), jnp.float32))
    outs = pl.pallas_call(
        body,
        grid=(N2 // RA,),
        in_specs=in_specs,
        out_specs=out_specs,
        out_shape=out_shape,
    )(y, res, s1, s2, w.reshape(1, GAT_DIM), b.reshape(1, GAT_DIM),
      ms.reshape(1, GAT_DIM))
    return outs if with_pool else (outs[0], None)


def _cls_body(p_ref, w_ref, b_ref, out_ref):
    pooled = p_ref[:, 0, :] * (1.0 / NUM_NODES)
    out_ref[...] = jnp.sum(pooled * w_ref[...], axis=1, keepdims=True) + b_ref[...]


def _cls_call(psum, clsW, clsb):
    return pl.pallas_call(
        _cls_body,
        out_shape=jax.ShapeDtypeStruct((B, 1), jnp.float32),
    )(psum, clsW.reshape(1, GAT_DIM), clsb.reshape(1, 1))


def kernel(x, edge_index, params):
    xs2d = jnp.transpose(x, (0, 2, 1)).reshape(N2, T)
    h = _lstm_call(xs2d, params)

    src2d = edge_index[0].reshape(E // CH, CH).astype(jnp.int32)
    dst2d = edge_index[1].reshape(E // CH, CH).astype(jnp.int32)

    psum = None
    for g in (1, 2, 3):
        hp, asp, adp = _proj_call(h, params['gat%d_W' % g],
                                  params['gat%d_att_src' % g],
                                  params['gat%d_att_dst' % g])
        o = _edge_call(hp, asp, adp, src2d, dst2d)
        y, s1, s2 = _postA_call(o, params['gat%d_bias' % g])
        h, psum = _postB_call(y, h, s1, s2, params['norm%d_w' % g],
                              params['norm%d_b' % g], params['norm%d_ms' % g],
                              with_pool=(g == 3))
    return _cls_call(psum, params['cls_W'], params['cls_b'])


# speedup vs baseline: 1.1587x; 1.1587x over previous
import functools

import jax
import jax.numpy as jnp
from jax import lax
from jax.experimental import pallas as pl
from jax.experimental.pallas import tpu as pltpu
from jax.experimental.pallas import tpu_sc as plsc

B = 2
T = 8
NUM_NODES = 10000
E = 320000
H = 64
HEADS = 4
GAT_DIM = 128
N2 = B * NUM_NODES


RL = 1000


def _lstm_body(x_ref, wih0f, whh0f, b0f, b0f2, wih0b, whh0b, b0b, b0b2,
               wih1f, whh1f, b1f, b1f2, wih1b, whh1b, b1b, b1b2, out_ref):
    x = x_ref[...]

    def cell(pre, h, c, whhT, bih, bhh):
        gates = ((pre + jnp.dot(h, whhT[...], preferred_element_type=jnp.float32))
                 + bih[...]) + bhh[...]
        i = gates[:, 0:H]
        f = gates[:, H:2 * H]
        g = gates[:, 2 * H:3 * H]
        o = gates[:, 3 * H:4 * H]
        c = jax.nn.sigmoid(f) * c + jax.nn.sigmoid(i) * jnp.tanh(g)
        h = jax.nn.sigmoid(o) * jnp.tanh(c)
        return h, c

    z = jnp.zeros((x.shape[0], H), jnp.float32)

    hf0 = []
    h, c = z, z
    for t in range(T):
        h, c = cell(x[:, t:t + 1] * wih0f[...], h, c, whh0f, b0f, b0f2)
        hf0.append(h)
    hb0 = [None] * T
    h, c = z, z
    for t in range(T - 1, -1, -1):
        h, c = cell(x[:, t:t + 1] * wih0b[...], h, c, whh0b, b0b, b0b2)
        hb0[t] = h

    zs = [jnp.concatenate([hf0[t], hb0[t]], axis=1) for t in range(T)]

    sumf = z
    h, c = z, z
    for t in range(T):
        h, c = cell(jnp.dot(zs[t], wih1f[...], preferred_element_type=jnp.float32),
                    h, c, whh1f, b1f, b1f2)
        sumf = sumf + h
    hb1 = [None] * T
    h, c = z, z
    for t in range(T - 1, -1, -1):
        h, c = cell(jnp.dot(zs[t], wih1b[...], preferred_element_type=jnp.float32),
                    h, c, whh1b, b1b, b1b2)
        hb1[t] = h
    sumb = z
    for t in range(T):
        sumb = sumb + hb1[t]

    out_ref[...] = jnp.concatenate([sumf, sumb], axis=1) * (1.0 / T)


def _lstm_call(xs2d, wp):
    full = lambda shp: pl.BlockSpec(shp, lambda i: (0, 0))
    specs = [pl.BlockSpec((RL, T), lambda i: (i, 0))]
    for l in (0, 1):
        for d in ('f', 'b'):
            ih_shape = (1, 4 * H) if l == 0 else (2 * H, 4 * H)
            specs += [full(ih_shape), full((H, 4 * H)), full((1, 4 * H)),
                      full((1, 4 * H))]
    args = [xs2d]
    for l in (0, 1):
        for d in ('f', 'b'):
            pre = 'lstm%d%s_' % (l, d)
            args += [wp[pre + 'Wih'].T, wp[pre + 'Whh'].T,
                     wp[pre + 'bih'][None, :], wp[pre + 'bhh'][None, :]]
    return pl.pallas_call(
        _lstm_body,
        grid=(N2 // RL,),
        in_specs=specs,
        out_specs=pl.BlockSpec((RL, GAT_DIM), lambda i: (i, 0)),
        out_shape=jax.ShapeDtypeStruct((N2, GAT_DIM), jnp.float32),
    )(*args)


RP = 2000


def _proj_body(h_ref, wT_ref, asf_ref, adf_ref, hp_ref, asp_ref, adp_ref):
    hp = jnp.dot(h_ref[...], wT_ref[...], preferred_element_type=jnp.float32,
                 precision=lax.Precision.HIGHEST)
    hp_ref[...] = hp
    ts = hp * asf_ref[...]
    td = hp * adf_ref[...]
    z = jnp.zeros((hp.shape[0], 12), jnp.float32)
    cols = [jnp.sum(ts[:, 32 * k:32 * (k + 1)], axis=1, keepdims=True)
            for k in range(HEADS)] + [z]
    asp_ref[...] = jnp.concatenate(cols, axis=1)
    cols = [jnp.sum(td[:, 32 * k:32 * (k + 1)], axis=1, keepdims=True)
            for k in range(HEADS)] + [z]
    adp_ref[...] = jnp.concatenate(cols, axis=1)


def _proj_call(h, W, att_src, att_dst):
    full = lambda shp: pl.BlockSpec(shp, lambda i: (0, 0))
    return pl.pallas_call(
        _proj_body,
        grid=(N2 // RP,),
        in_specs=[pl.BlockSpec((RP, GAT_DIM), lambda i: (i, 0)),
                  full((GAT_DIM, GAT_DIM)), full((1, GAT_DIM)), full((1, GAT_DIM))],
        out_specs=[pl.BlockSpec((RP, GAT_DIM), lambda i: (i, 0)),
                   pl.BlockSpec((RP, 16), lambda i: (i, 0)),
                   pl.BlockSpec((RP, 16), lambda i: (i, 0))],
        out_shape=[jax.ShapeDtypeStruct((N2, GAT_DIM), jnp.float32),
                   jax.ShapeDtypeStruct((N2, 16), jnp.float32),
                   jax.ShapeDtypeStruct((N2, 16), jnp.float32)],
    )(h, W.T, att_src.reshape(1, GAT_DIM), att_dst.reshape(1, GAT_DIM))


CH = 80
SUP = 8
NROW = E // CH
NSUPT = NROW // SUP
ZB = 80
NZB = NUM_NODES // ZB
ACC_W = 144


def _edge_sc_body(hp0, hp1, as0, as1, ad0, ad1, src2d, dst2d, out_hbm,
                  srcb, dstb, gbuf, asb, adb, gbuf2, asb2, adb2, rows,
                  acc, gsem, asem, adsem, gsem2, asem2, adsem2):
    c = lax.axis_index("c")
    s = lax.axis_index("s")
    z16 = jnp.zeros((16,), jnp.float32)

    @pl.loop(0, ZB)
    def _zero(r):
        for f in range(ACC_W // 16):
            rows[r, pl.ds(16 * f, 16)] = z16

    @pl.loop(0, (NZB + 15) // 16)
    def _zcp(q):
        b = s + 16 * q

        @pl.when(b < NZB)
        def _():
            pltpu.sync_copy(rows, acc.at[pl.ds(ZB * b, ZB)])

    plsc.subcore_barrier()

    def compute(gbuf, asb, adb):
        @pl.loop(0, CH, unroll=4)
        def _edge(e):
            w = asb[e, pl.ds(0, 16)] + adb[e, pl.ds(0, 16)]
            w = jnp.where(w >= 0.0, w, w * 0.2)
            w = jnp.exp(w)
            rows[e, pl.ds(GAT_DIM, 16)] = w
            ws = (w[0], w[1], w[2], w[3])
            for f in range(8):
                rows[e, pl.ds(16 * f, 16)] = gbuf[e, pl.ds(16 * f, 16)] * ws[f // 2]

    def fire(hp, a_s, a_d, i, gbuf, asb, adb, gsem, asem, adsem):
        pltpu.async_copy(hp.at[srcb.at[i]], gbuf, gsem)
        pltpu.async_copy(a_s.at[srcb.at[i]], asb, asem)
        pltpu.async_copy(a_d.at[dstb.at[i]], adb, adsem)

    def drain(hp, a_s, a_d, i, gbuf, asb, adb, gsem, asem, adsem):
        pltpu.make_async_copy(hp.at[srcb.at[i]], gbuf, gsem).wait()
        pltpu.make_async_copy(a_s.at[srcb.at[i]], asb, asem).wait()
        pltpu.make_async_copy(a_d.at[dstb.at[i]], adb, adsem).wait()

    def step(hp, a_s, a_d, i, cur, nxt):
        drain(hp, a_s, a_d, i, *cur)

        @pl.when(i < SUP - 1)
        def _():
            fire(hp, a_s, a_d, i + 1, *nxt)

        compute(cur[0], cur[1], cur[2])
        pltpu.sync_copy(rows, acc.at[dstb.at[i]], add=True)

    buf0 = (gbuf, asb, adb, gsem, asem, adsem)
    buf1 = (gbuf2, asb2, adb2, gsem2, asem2, adsem2)

    @pl.loop(0, (NSUPT + 15) // 16)
    def _sup(j):
        u = s + 16 * j

        @pl.when(u < NSUPT)
        def _():
            row0 = SUP * u
            pltpu.sync_copy(src2d.at[pl.ds(row0, SUP)], srcb)
            pltpu.sync_copy(dst2d.at[pl.ds(row0, SUP)], dstb)

            @pl.when(c == 0)
            def _():
                fire(hp0, as0, ad0, 0, *buf0)

            @pl.when(c == 1)
            def _():
                fire(hp1, as1, ad1, 0, *buf0)

            @pl.loop(0, SUP)
            def _chunk(i):
                @pl.when(jnp.logical_and(c == 0, i % 2 == 0))
                def _():
                    step(hp0, as0, ad0, i, buf0, buf1)

                @pl.when(jnp.logical_and(c == 0, i % 2 == 1))
                def _():
                    step(hp0, as0, ad0, i, buf1, buf0)

                @pl.when(jnp.logical_and(c == 1, i % 2 == 0))
                def _():
                    step(hp1, as1, ad1, i, buf0, buf1)

                @pl.when(jnp.logical_and(c == 1, i % 2 == 1))
                def _():
                    step(hp1, as1, ad1, i, buf1, buf0)

    plsc.subcore_barrier()

    @pl.loop(0, (NZB + 15) // 16)
    def _wout(q):
        b = s + 16 * q

        @pl.when(b < NZB)
        def _():
            pltpu.sync_copy(acc.at[pl.ds(ZB * b, ZB)],
                            out_hbm.at[pl.ds(c * NUM_NODES + ZB * b, ZB)])


def _edge_call(hp, asp, adp, src2d, dst2d):
    mesh = plsc.VectorSubcoreMesh(core_axis_name="c", subcore_axis_name="s",
                                  num_cores=2, num_subcores=16)
    fn = pl.kernel(
        _edge_sc_body,
        out_type=jax.ShapeDtypeStruct((N2, ACC_W), jnp.float32),
        mesh=mesh,
        compiler_params=pltpu.CompilerParams(use_tc_tiling_on_sc=False),
        scratch_types=[
            pltpu.VMEM((SUP, CH), jnp.int32),
            pltpu.VMEM((SUP, CH), jnp.int32),
            pltpu.VMEM((CH, GAT_DIM), jnp.float32),
            pltpu.VMEM((CH, 16), jnp.float32),
            pltpu.VMEM((CH, 16), jnp.float32),
            pltpu.VMEM((CH, GAT_DIM), jnp.float32),
            pltpu.VMEM((CH, 16), jnp.float32),
            pltpu.VMEM((CH, 16), jnp.float32),
            pltpu.VMEM((CH, ACC_W), jnp.float32),
            pltpu.VMEM_SHARED((NUM_NODES, ACC_W), jnp.float32),
            pltpu.SemaphoreType.DMA,
            pltpu.SemaphoreType.DMA,
            pltpu.SemaphoreType.DMA,
            pltpu.SemaphoreType.DMA,
            pltpu.SemaphoreType.DMA,
            pltpu.SemaphoreType.DMA,
        ],
    )
    return fn(hp[:NUM_NODES], hp[NUM_NODES:], asp[:NUM_NODES], asp[NUM_NODES:],
              adp[:NUM_NODES], adp[NUM_NODES:], src2d, dst2d)


RA = 2000
BPB = NUM_NODES // RA


def _postA_body(o_ref, bias_ref, y_ref, s1_ref, s2_ref):
    o = o_ref[...]
    cols = []
    for k in range(HEADS):
        cols.append(o[:, 32 * k:32 * (k + 1)]
                    / (o[:, GAT_DIM + k:GAT_DIM + k + 1] + 1e-16))
    y = jnp.concatenate(cols, axis=1) + bias_ref[...]
    y_ref[...] = y

    @pl.when(pl.program_id(0) % BPB == 0)
    def _():
        s1_ref[...] = jnp.zeros_like(s1_ref)
        s2_ref[...] = jnp.zeros_like(s2_ref)

    s1_ref[...] += jnp.sum(y, axis=0, keepdims=True)[None]
    s2_ref[...] += jnp.sum(y * y, axis=0, keepdims=True)[None]


def _postA_call(o, bias):
    full = lambda shp: pl.BlockSpec(shp, lambda i: (0, 0))
    return pl.pallas_call(
        _postA_body,
        grid=(N2 // RA,),
        in_specs=[pl.BlockSpec((RA, ACC_W), lambda i: (i, 0)), full((1, GAT_DIM))],
        out_specs=[pl.BlockSpec((RA, GAT_DIM), lambda i: (i, 0)),
                   pl.BlockSpec((1, 1, GAT_DIM), lambda i: (i // BPB, 0, 0)),
                   pl.BlockSpec((1, 1, GAT_DIM), lambda i: (i // BPB, 0, 0))],
        out_shape=[jax.ShapeDtypeStruct((N2, GAT_DIM), jnp.float32),
                   jax.ShapeDtypeStruct((B, 1, GAT_DIM), jnp.float32),
                   jax.ShapeDtypeStruct((B, 1, GAT_DIM), jnp.float32)],
    )(o, bias.reshape(1, GAT_DIM))


def _postB_compute(y_ref, res_ref, s1_ref, s2_ref, w_ref, b_ref, ms_ref):
    inv_n = 1.0 / NUM_NODES
    mean = s1_ref[0] * inv_n
    mm = mean * ms_ref[...]
    var = s2_ref[0] * inv_n - 2.0 * mm * mean + mm * mm
    inv = 1.0 / jnp.sqrt(var + 1e-5)
    xc = y_ref[...] - mm
    return jax.nn.relu(w_ref[...] * xc * inv + b_ref[...] + res_ref[...])


def _postB_body(y_ref, res_ref, s1_ref, s2_ref, w_ref, b_ref, ms_ref, out_ref):
    out_ref[...] = _postB_compute(y_ref, res_ref, s1_ref, s2_ref, w_ref, b_ref, ms_ref)


def _postB_pool_body(y_ref, res_ref, s1_ref, s2_ref, w_ref, b_ref, ms_ref,
                     out_ref, p_ref):
    h = _postB_compute(y_ref, res_ref, s1_ref, s2_ref, w_ref, b_ref, ms_ref)
    out_ref[...] = h

    @pl.when(pl.program_id(0) % BPB == 0)
    def _():
        p_ref[...] = jnp.zeros_like(p_ref)

    p_ref[...] += jnp.sum(h, axis=0, keepdims=True)[None]


def _postB_call(y, res, s1, s2, w, b, ms, with_pool):
    full = lambda shp: pl.BlockSpec(shp, lambda i: (0, 0))
    in_specs = [pl.BlockSpec((RA, GAT_DIM), lambda i: (i, 0)),
                pl.BlockSpec((RA, GAT_DIM), lambda i: (i, 0)),
                pl.BlockSpec((1, 1, GAT_DIM), lambda i: (i // BPB, 0, 0)),
                pl.BlockSpec((1, 1, GAT_DIM), lambda i: (i // BPB, 0, 0)),
                full((1, GAT_DIM)), full((1, GAT_DIM)), full((1, GAT_DIM))]
    out_specs = [pl.BlockSpec((RA, GAT_DIM), lambda i: (i, 0))]
    out_shape = [jax.ShapeDtypeStruct((N2, GAT_DIM), jnp.float32)]
    body = _postB_body
    if with_pool:
        body = _postB_pool_body
        out_specs.append(pl.BlockSpec((1, 1, GAT_DIM), lambda i: (i // BPB, 0, 0)))
        out_shape.append(jax.ShapeDtypeStruct((B, 1, GAT_DIM), jnp.float32))
    outs = pl.pallas_call(
        body,
        grid=(N2 // RA,),
        in_specs=in_specs,
        out_specs=out_specs,
        out_shape=out_shape,
    )(y, res, s1, s2, w.reshape(1, GAT_DIM), b.reshape(1, GAT_DIM),
      ms.reshape(1, GAT_DIM))
    return outs if with_pool else (outs[0], None)


def _cls_body(p_ref, w_ref, b_ref, out_ref):
    pooled = p_ref[:, 0, :] * (1.0 / NUM_NODES)
    out_ref[...] = jnp.sum(pooled * w_ref[...], axis=1, keepdims=True) + b_ref[...]


def _cls_call(psum, clsW, clsb):
    return pl.pallas_call(
        _cls_body,
        out_shape=jax.ShapeDtypeStruct((B, 1), jnp.float32),
    )(psum, clsW.reshape(1, GAT_DIM), clsb.reshape(1, 1))


def kernel(x, edge_index, params):
    xs2d = jnp.transpose(x, (0, 2, 1)).reshape(N2, T)
    h = _lstm_call(xs2d, params)

    src2d = edge_index[0].reshape(E // CH, CH).astype(jnp.int32)
    dst2d = edge_index[1].reshape(E // CH, CH).astype(jnp.int32)

    psum = None
    for g in (1, 2, 3):
        hp, asp, adp = _proj_call(h, params['gat%d_W' % g],
                                  params['gat%d_att_src' % g],
                                  params['gat%d_att_dst' % g])
        o = _edge_call(hp, asp, adp, src2d, dst2d)
        y, s1, s2 = _postA_call(o, params['gat%d_bias' % g])
        h, psum = _postB_call(y, h, s1, s2, params['norm%d_w' % g],
                              params['norm%d_b' % g], params['norm%d_ms' % g],
                              with_pool=(g == 3))
    return _cls_call(psum, params['cls_W'], params['cls_b'])
